# baseline (speedup 1.0000x reference)
import functools
import jax
import jax.numpy as jnp
from jax import lax
from jax.experimental import pallas as pl
from jax.experimental.pallas import tpu as pltpu
from jax.experimental.pallas import tpu_sc as plsc

N_ENT = 10000
N_HE = 60000
N_INC = 320000
D = 128
BATCH = 4096
NTY = 294

NC = 2
NS = 16
LANE = 16

NCH = D // LANE
ZROW = N_ENT + NTY
NT_PAD = 10368
NT_TPW = NT_PAD // NS

S1_PAD = 16 * 40960
S1_TPW = S1_PAD // NS // 128
S2_PAD = 16 * 20480
S2_TPW = S2_PAD // NS // 128
IDX_CH = 80

XE_ROWS = 60160
XV_ROWS = 10112
XV_TPW = XV_ROWS // NS
XE_PAD_ROW = 60100
XV_PAD_ROW = 10050
ZB_ROWS = 376


def _lorentz_tc(emb_ref, w_ref, b_ref, s_ref, x_ref):
    x = jax.lax.dot_general(emb_ref[...], w_ref[...],
                            dimension_numbers=(((1,), (1,)), ((), ())),
                            preferred_element_type=jnp.float32)
    x = x + b_ref[...]
    col = lax.broadcasted_iota(jnp.int32, x.shape, 1)
    is0 = col == 0
    time = jax.nn.sigmoid(x[:, :1]) * jnp.exp(s_ref[0, 0]) + 1.1
    xs = jnp.where(is0, 0.0, x)
    denom = jnp.maximum(jnp.sum(xs * xs, axis=-1, keepdims=True), 1e-8)
    scale = (time * time - 1.0) / denom
    x_ref[...] = jnp.where(is0, time, x * jnp.sqrt(scale))


def _logmap_tc(xv_ref, x_ref, eps_ref, out_ref):
    xc = eps_ref[0, 0] * xv_ref[...] + x_ref[...]
    col = lax.broadcasted_iota(jnp.int32, xc.shape, 1)
    row = lax.broadcasted_iota(jnp.int32, xc.shape, 0)
    is0 = col == 0
    y = jnp.where(is0, 0.0, xc)
    y_norm = jnp.maximum(jnp.sqrt(jnp.sum(y * y, axis=-1, keepdims=True)), 1e-8)
    theta = jnp.maximum(xc[:, :1], 1.0 + 1e-7)
    acosh = jnp.log(theta + jnp.sqrt(theta * theta - 1.0))
    out = jnp.where(is0, 0.0, xc * (acosh / y_norm))
    out_ref[...] = jnp.where(row == 0, 1.0, out)


def _final_tc(g6_ref, gr_ref, out_ref):
    p = gr_ref[...]
    for j in range(6):
        p = p * g6_ref[j]
    out_ref[...] = jnp.sum(p, axis=-1).reshape(out_ref.shape)


GRP = 8


def _sc_two_stage(t_hbm, i1_hbm, s1_hbm, e2_hbm, v2_hbm, out_hbm,
                  idx_v, seg_v, rows_v, zbuf, gsem, ssem,
                  tbl_s, xe_s, xv_s):
    cid = lax.axis_index("c")
    sid = lax.axis_index("s")

    def _seg_pass(src, idx_hbm, seg_hbm, dst, tpw):
        @pl.loop(0, tpw // IDX_CH)
        def _(b):
            pltpu.sync_copy(
                idx_hbm.at[pl.ds(sid * tpw + b * IDX_CH, IDX_CH)], idx_v)
            pltpu.sync_copy(
                seg_hbm.at[pl.ds(sid * tpw + b * IDX_CH, IDX_CH)], seg_v)

            @pl.loop(0, IDX_CH // GRP)
            def _(g):
                gh = [pltpu.async_copy(src.at[idx_v.at[g * GRP + k]],
                                       rows_v.at[k], gsem)
                      for k in range(GRP)]
                sh = []
                for k in range(GRP):
                    gh[k].wait()
                    sh.append(pltpu.async_copy(
                        rows_v.at[k], dst.at[seg_v.at[g * GRP + k]],
                        ssem, add=True))
                for h in sh:
                    h.wait()

    @pl.loop(0, ZB_ROWS)
    def _(i):
        zbuf[i, :] = jnp.zeros((LANE,), jnp.float32)

    @pl.loop(0, NCH // NC)
    def _(k):
        c = cid * (NCH // NC) + k

        @pl.loop(0, 10)
        def _(z):
            pltpu.sync_copy(zbuf, xe_s.at[pl.ds(sid * 3760 + z * ZB_ROWS, ZB_ROWS)])
        pltpu.sync_copy(zbuf, xv_s.at[pl.ds(sid * XV_TPW, ZB_ROWS)])
        pltpu.sync_copy(zbuf.at[pl.ds(0, XV_TPW - ZB_ROWS)],
                        xv_s.at[pl.ds(sid * XV_TPW + ZB_ROWS, XV_TPW - ZB_ROWS)])
        pltpu.sync_copy(t_hbm.at[pl.ds(sid * NT_TPW, NT_TPW), c],
                        tbl_s.at[pl.ds(sid * NT_TPW, NT_TPW)])
        plsc.subcore_barrier()

        _seg_pass(tbl_s, i1_hbm, s1_hbm, xe_s, S1_TPW)
        plsc.subcore_barrier()

        _seg_pass(xe_s, e2_hbm, v2_hbm, xv_s, S2_TPW)
        plsc.subcore_barrier()

        pltpu.sync_copy(xv_s.at[pl.ds(sid * XV_TPW, XV_TPW)],
                        out_hbm.at[c].at[pl.ds(sid * XV_TPW, XV_TPW)])
        plsc.subcore_barrier()


def _sc_final_gather(ee_hbm, i6_hbm, rp_hbm, o6_hbm, or_hbm,
                     i6_v, rows_v):
    cid = lax.axis_index("c")
    sid = lax.axis_index("s")
    wid = sid * NC + cid

    pltpu.sync_copy(i6_hbm.at[wid], i6_v)

    @pl.loop(0, 6)
    def _(j):
        pltpu.sync_copy(ee_hbm.at[i6_v.at[j]], rows_v)
        pltpu.sync_copy(rows_v, o6_hbm.at[pl.ds((j * 32 + wid) * 128, 128)])

    pltpu.sync_copy(rp_hbm.at[i6_v.at[6]], rows_v)
    pltpu.sync_copy(rows_v, or_hbm.at[pl.ds(wid * 128, 128)])


@jax.jit
def _run(r_idx, e1_idx, e2_idx, e3_idx, e4_idx, e5_idx, e6_idx,
         V, E, ty, emb_E, emb_R, emb_ty, W_lin, b_lin, scale_lin, eps):
    f32 = jnp.float32

    X = pl.pallas_call(
        _lorentz_tc,
        out_shape=jax.ShapeDtypeStruct((N_ENT, D), f32),
    )(emb_E, W_lin, b_lin, jnp.reshape(scale_lin, (1, 1)).astype(f32))

    T = jnp.concatenate(
        [X, -emb_ty, jnp.zeros((NT_PAD - N_ENT - NTY, D), f32)], axis=0)
    Tc = T.reshape(NT_PAD, NCH, LANE)

    n1pad = S1_PAD - 2 * N_INC
    idx1 = jnp.concatenate(
        [V, N_ENT + ty, jnp.full((n1pad,), ZROW, jnp.int32)]).reshape(-1, 128)
    seg1 = jnp.concatenate(
        [E, E, jnp.zeros((n1pad,), jnp.int32)]).reshape(-1, 128)
    npad = S2_PAD - N_INC
    e2 = jnp.concatenate(
        [E, jnp.full((npad,), XE_PAD_ROW, jnp.int32)]).reshape(-1, 128)
    v2 = jnp.concatenate(
        [V, jnp.full((npad,), XV_PAD_ROW, jnp.int32)]).reshape(-1, 128)

    mesh = plsc.VectorSubcoreMesh(core_axis_name="c", subcore_axis_name="s")
    sc_params = pltpu.CompilerParams(use_tc_tiling_on_sc=False)
    xv_ch = pl.kernel(
        _sc_two_stage,
        out_type=jax.ShapeDtypeStruct((NCH, XV_ROWS, LANE), f32),
        mesh=mesh,
        compiler_params=sc_params,
        scratch_types=[
            pltpu.VMEM((IDX_CH, 128), jnp.int32),
            pltpu.VMEM((IDX_CH, 128), jnp.int32),
            pltpu.VMEM((GRP, 128, LANE), f32),
            pltpu.VMEM((ZB_ROWS, LANE), f32),
            pltpu.SemaphoreType.DMA,
            pltpu.SemaphoreType.DMA,
            pltpu.VMEM_SHARED((NT_PAD, LANE), f32),
            pltpu.VMEM_SHARED((XE_ROWS, LANE), f32),
            pltpu.VMEM_SHARED((XV_ROWS, LANE), f32),
        ],
    )(Tc, idx1, seg1, e2, v2)

    Xv = jnp.transpose(xv_ch, (1, 0, 2)).reshape(XV_ROWS, D)[:N_ENT]

    E_e = pl.pallas_call(
        _logmap_tc,
        out_shape=jax.ShapeDtypeStruct((N_ENT, D), f32),
    )(Xv, X, jnp.reshape(eps, (1, 1)).astype(f32))

    R_p = emb_R.at[0].set(jnp.ones((D,), f32))

    idx6 = jnp.transpose(
        jnp.stack([e1_idx, e2_idx, e3_idx, e4_idx, e5_idx, e6_idx]
                  ).reshape(6, 32, 128), (1, 0, 2))
    iall = jnp.concatenate(
        [idx6, r_idx.reshape(32, 1, 128),
         jnp.zeros((32, 1, 128), jnp.int32)], axis=1)
    g6, gr = pl.kernel(
        _sc_final_gather,
        out_type=(jax.ShapeDtypeStruct((6 * BATCH, D), f32),
                  jax.ShapeDtypeStruct((BATCH, D), f32)),
        mesh=mesh,
        scratch_types=[
            pltpu.VMEM((8, 128), jnp.int32),
            pltpu.VMEM((128, D), f32),
        ],
    )(E_e, iall, R_p)

    out = pl.pallas_call(
        _final_tc,
        out_shape=jax.ShapeDtypeStruct((BATCH // 128, 128), f32),
    )(g6.reshape(6, BATCH, D), gr)
    return out.reshape(BATCH)


def kernel(r_idx, e1_idx, e2_idx, e3_idx, e4_idx, e5_idx, e6_idx, ms, bs,
           V, E, ty, emb_E, emb_R, emb_ty, W_lin, b_lin, scale_lin, eps):
    del ms, bs
    return _run(r_idx, e1_idx, e2_idx, e3_idx, e4_idx, e5_idx, e6_idx,
                V, E, ty, emb_E, emb_R, emb_ty, W_lin, b_lin, scale_lin, eps)


# speedup vs baseline: 1.1123x; 1.1123x over previous
import functools
import jax
import jax.numpy as jnp
from jax import lax
from jax.experimental import pallas as pl
from jax.experimental.pallas import tpu as pltpu
from jax.experimental.pallas import tpu_sc as plsc

N_ENT = 10000
N_HE = 60000
N_INC = 320000
D = 128
BATCH = 4096
NTY = 294

NC = 2
NS = 16
LANE = 16

NCH = D // LANE
ZROW = N_ENT + NTY
NT_PAD = 10368
NT_TPW = NT_PAD // NS

S1_PAD = 16 * 40960
S1_TPW = S1_PAD // NS // 128
S2_PAD = 16 * 20480
S2_TPW = S2_PAD // NS // 128
IDX_CH = 80

XE_ROWS = 60160
XV_ROWS = 10112
XV_TPW = XV_ROWS // NS
XE_PAD_ROW = 60100
XV_PAD_ROW = 10050
ZB_ROWS = 376


def _lorentz_tc(emb_ref, w_ref, b_ref, s_ref, x_ref):
    x = jax.lax.dot_general(emb_ref[...], w_ref[...],
                            dimension_numbers=(((1,), (1,)), ((), ())),
                            preferred_element_type=jnp.float32)
    x = x + b_ref[...]
    col = lax.broadcasted_iota(jnp.int32, x.shape, 1)
    is0 = col == 0
    time = jax.nn.sigmoid(x[:, :1]) * jnp.exp(s_ref[0, 0]) + 1.1
    xs = jnp.where(is0, 0.0, x)
    denom = jnp.maximum(jnp.sum(xs * xs, axis=-1, keepdims=True), 1e-8)
    scale = (time * time - 1.0) / denom
    x_ref[...] = jnp.where(is0, time, x * jnp.sqrt(scale))


def _logmap_tc(xv_ref, x_ref, eps_ref, out_ref):
    xc = eps_ref[0, 0] * xv_ref[...] + x_ref[...]
    col = lax.broadcasted_iota(jnp.int32, xc.shape, 1)
    row = lax.broadcasted_iota(jnp.int32, xc.shape, 0)
    is0 = col == 0
    y = jnp.where(is0, 0.0, xc)
    y_norm = jnp.maximum(jnp.sqrt(jnp.sum(y * y, axis=-1, keepdims=True)), 1e-8)
    theta = jnp.maximum(xc[:, :1], 1.0 + 1e-7)
    acosh = jnp.log(theta + jnp.sqrt(theta * theta - 1.0))
    out = jnp.where(is0, 0.0, xc * (acosh / y_norm))
    out_ref[...] = jnp.where(row == 0, 1.0, out)


def _final_tc(g6_ref, gr_ref, out_ref):
    p = gr_ref[...]
    for j in range(6):
        p = p * g6_ref[j]
    out_ref[...] = jnp.sum(p, axis=-1).reshape(out_ref.shape)


GRP = 4


def _sc_two_stage(t_hbm, i1_hbm, s1_hbm, e2_hbm, v2_hbm, out_hbm,
                  idx_v, seg_v, rows_v, zbuf, gsem, ssem,
                  tbl_s, xe_s, xv_s):
    cid = lax.axis_index("c")
    sid = lax.axis_index("s")

    def _seg_pass(src, idx_hbm, seg_hbm, dst, tpw):
        @pl.loop(0, tpw // IDX_CH)
        def _(b):
            pltpu.sync_copy(
                idx_hbm.at[pl.ds(sid * tpw + b * IDX_CH, IDX_CH)], idx_v)
            pltpu.sync_copy(
                seg_hbm.at[pl.ds(sid * tpw + b * IDX_CH, IDX_CH)], seg_v)

            sh_buf = [[], []]
            for g in range(IDX_CH // GRP):
                buf = g % 2
                for h in sh_buf[buf]:
                    h.wait()
                gh = [pltpu.async_copy(src.at[idx_v.at[g * GRP + k]],
                                       rows_v.at[buf, k], gsem)
                      for k in range(GRP)]
                sh = []
                for k in range(GRP):
                    gh[k].wait()
                    sh.append(pltpu.async_copy(
                        rows_v.at[buf, k], dst.at[seg_v.at[g * GRP + k]],
                        ssem, add=True))
                sh_buf[buf] = sh
            for sh in sh_buf:
                for h in sh:
                    h.wait()

    @pl.loop(0, ZB_ROWS)
    def _(i):
        zbuf[i, :] = jnp.zeros((LANE,), jnp.float32)

    @pl.loop(0, NCH // NC)
    def _(k):
        c = cid * (NCH // NC) + k

        @pl.loop(0, 10)
        def _(z):
            pltpu.sync_copy(zbuf, xe_s.at[pl.ds(sid * 3760 + z * ZB_ROWS, ZB_ROWS)])
        pltpu.sync_copy(zbuf, xv_s.at[pl.ds(sid * XV_TPW, ZB_ROWS)])
        pltpu.sync_copy(zbuf.at[pl.ds(0, XV_TPW - ZB_ROWS)],
                        xv_s.at[pl.ds(sid * XV_TPW + ZB_ROWS, XV_TPW - ZB_ROWS)])
        pltpu.sync_copy(t_hbm.at[pl.ds(sid * NT_TPW, NT_TPW), c],
                        tbl_s.at[pl.ds(sid * NT_TPW, NT_TPW)])
        plsc.subcore_barrier()

        _seg_pass(tbl_s, i1_hbm, s1_hbm, xe_s, S1_TPW)
        plsc.subcore_barrier()

        _seg_pass(xe_s, e2_hbm, v2_hbm, xv_s, S2_TPW)
        plsc.subcore_barrier()

        pltpu.sync_copy(xv_s.at[pl.ds(sid * XV_TPW, XV_TPW)],
                        out_hbm.at[c].at[pl.ds(sid * XV_TPW, XV_TPW)])
        plsc.subcore_barrier()


def _sc_final_gather(ee_hbm, i6_hbm, rp_hbm, o6_hbm, or_hbm,
                     i6_v, rows_v):
    cid = lax.axis_index("c")
    sid = lax.axis_index("s")
    wid = sid * NC + cid

    pltpu.sync_copy(i6_hbm.at[wid], i6_v)

    @pl.loop(0, 6)
    def _(j):
        pltpu.sync_copy(ee_hbm.at[i6_v.at[j]], rows_v)
        pltpu.sync_copy(rows_v, o6_hbm.at[pl.ds((j * 32 + wid) * 128, 128)])

    pltpu.sync_copy(rp_hbm.at[i6_v.at[6]], rows_v)
    pltpu.sync_copy(rows_v, or_hbm.at[pl.ds(wid * 128, 128)])


@jax.jit
def _run(r_idx, e1_idx, e2_idx, e3_idx, e4_idx, e5_idx, e6_idx,
         V, E, ty, emb_E, emb_R, emb_ty, W_lin, b_lin, scale_lin, eps):
    f32 = jnp.float32

    X = pl.pallas_call(
        _lorentz_tc,
        out_shape=jax.ShapeDtypeStruct((N_ENT, D), f32),
    )(emb_E, W_lin, b_lin, jnp.reshape(scale_lin, (1, 1)).astype(f32))

    T = jnp.concatenate(
        [X, -emb_ty, jnp.zeros((NT_PAD - N_ENT - NTY, D), f32)], axis=0)
    Tc = T.reshape(NT_PAD, NCH, LANE)

    n1pad = S1_PAD - 2 * N_INC
    idx1 = jnp.concatenate(
        [V, N_ENT + ty, jnp.full((n1pad,), ZROW, jnp.int32)]).reshape(-1, 128)
    seg1 = jnp.concatenate(
        [E, E, jnp.zeros((n1pad,), jnp.int32)]).reshape(-1, 128)
    npad = S2_PAD - N_INC
    e2 = jnp.concatenate(
        [E, jnp.full((npad,), XE_PAD_ROW, jnp.int32)]).reshape(-1, 128)
    v2 = jnp.concatenate(
        [V, jnp.full((npad,), XV_PAD_ROW, jnp.int32)]).reshape(-1, 128)

    mesh = plsc.VectorSubcoreMesh(core_axis_name="c", subcore_axis_name="s")
    sc_params = pltpu.CompilerParams(use_tc_tiling_on_sc=False)
    xv_ch = pl.kernel(
        _sc_two_stage,
        out_type=jax.ShapeDtypeStruct((NCH, XV_ROWS, LANE), f32),
        mesh=mesh,
        compiler_params=sc_params,
        scratch_types=[
            pltpu.VMEM((IDX_CH, 128), jnp.int32),
            pltpu.VMEM((IDX_CH, 128), jnp.int32),
            pltpu.VMEM((2, GRP, 128, LANE), f32),
            pltpu.VMEM((ZB_ROWS, LANE), f32),
            pltpu.SemaphoreType.DMA,
            pltpu.SemaphoreType.DMA,
            pltpu.VMEM_SHARED((NT_PAD, LANE), f32),
            pltpu.VMEM_SHARED((XE_ROWS, LANE), f32),
            pltpu.VMEM_SHARED((XV_ROWS, LANE), f32),
        ],
    )(Tc, idx1, seg1, e2, v2)

    Xv = jnp.transpose(xv_ch, (1, 0, 2)).reshape(XV_ROWS, D)[:N_ENT]

    E_e = pl.pallas_call(
        _logmap_tc,
        out_shape=jax.ShapeDtypeStruct((N_ENT, D), f32),
    )(Xv, X, jnp.reshape(eps, (1, 1)).astype(f32))

    R_p = emb_R.at[0].set(jnp.ones((D,), f32))

    idx6 = jnp.transpose(
        jnp.stack([e1_idx, e2_idx, e3_idx, e4_idx, e5_idx, e6_idx]
                  ).reshape(6, 32, 128), (1, 0, 2))
    iall = jnp.concatenate(
        [idx6, r_idx.reshape(32, 1, 128),
         jnp.zeros((32, 1, 128), jnp.int32)], axis=1)
    g6, gr = pl.kernel(
        _sc_final_gather,
        out_type=(jax.ShapeDtypeStruct((6 * BATCH, D), f32),
                  jax.ShapeDtypeStruct((BATCH, D), f32)),
        mesh=mesh,
        scratch_types=[
            pltpu.VMEM((8, 128), jnp.int32),
            pltpu.VMEM((128, D), f32),
        ],
    )(E_e, iall, R_p)

    out = pl.pallas_call(
        _final_tc,
        out_shape=jax.ShapeDtypeStruct((BATCH // 128, 128), f32),
    )(g6.reshape(6, BATCH, D), gr)
    return out.reshape(BATCH)


def kernel(r_idx, e1_idx, e2_idx, e3_idx, e4_idx, e5_idx, e6_idx, ms, bs,
           V, E, ty, emb_E, emb_R, emb_ty, W_lin, b_lin, scale_lin, eps):
    del ms, bs
    return _run(r_idx, e1_idx, e2_idx, e3_idx, e4_idx, e5_idx, e6_idx,
                V, E, ty, emb_E, emb_R, emb_ty, W_lin, b_lin, scale_lin, eps)


# speedup vs baseline: 1.1380x; 1.0231x over previous
import functools
import jax
import jax.numpy as jnp
from jax import lax
from jax.experimental import pallas as pl
from jax.experimental.pallas import tpu as pltpu
from jax.experimental.pallas import tpu_sc as plsc

N_ENT = 10000
N_HE = 60000
N_INC = 320000
D = 128
BATCH = 4096
NTY = 294

NC = 2
NS = 16
LANE = 16

NCH = D // LANE
ZROW = N_ENT + NTY
NT_PAD = 10368
NT_TPW = NT_PAD // NS

S1_PAD = 16 * 40960
S1_TPW = S1_PAD // NS // 128
S2_PAD = 16 * 20480
S2_TPW = S2_PAD // NS // 128
IDX_CH = 80

XE_ROWS = 60160
XV_ROWS = 10112
XV_TPW = XV_ROWS // NS
XE_PAD_ROW = 60100
XV_PAD_ROW = 10050
ZB_ROWS = 376


def _lorentz_tc(emb_ref, w_ref, b_ref, s_ref, x_ref):
    x = jax.lax.dot_general(emb_ref[...], w_ref[...],
                            dimension_numbers=(((1,), (1,)), ((), ())),
                            preferred_element_type=jnp.float32)
    x = x + b_ref[...]
    col = lax.broadcasted_iota(jnp.int32, x.shape, 1)
    is0 = col == 0
    time = jax.nn.sigmoid(x[:, :1]) * jnp.exp(s_ref[0, 0]) + 1.1
    xs = jnp.where(is0, 0.0, x)
    denom = jnp.maximum(jnp.sum(xs * xs, axis=-1, keepdims=True), 1e-8)
    scale = (time * time - 1.0) / denom
    x_ref[...] = jnp.where(is0, time, x * jnp.sqrt(scale))


def _logmap_tc(xv_ref, x_ref, eps_ref, out_ref):
    xc = eps_ref[0, 0] * xv_ref[...] + x_ref[...]
    col = lax.broadcasted_iota(jnp.int32, xc.shape, 1)
    row = lax.broadcasted_iota(jnp.int32, xc.shape, 0)
    is0 = col == 0
    y = jnp.where(is0, 0.0, xc)
    y_norm = jnp.maximum(jnp.sqrt(jnp.sum(y * y, axis=-1, keepdims=True)), 1e-8)
    theta = jnp.maximum(xc[:, :1], 1.0 + 1e-7)
    acosh = jnp.log(theta + jnp.sqrt(theta * theta - 1.0))
    out = jnp.where(is0, 0.0, xc * (acosh / y_norm))
    out_ref[...] = jnp.where(row == 0, 1.0, out)


def _final_tc(g6_ref, gr_ref, out_ref):
    p = gr_ref[...]
    for j in range(6):
        p = p * g6_ref[j]
    out_ref[...] = jnp.sum(p, axis=-1).reshape(out_ref.shape)


GRP = 5


def _sc_two_stage(t_hbm, i1_hbm, s1_hbm, e2_hbm, v2_hbm, out_hbm,
                  idx_v, seg_v, rows_v, zbuf, gsem, ssem,
                  tbl_s, xe_s, xv_s):
    cid = lax.axis_index("c")
    sid = lax.axis_index("s")

    def _seg_pass(src, idx_hbm, seg_hbm, dst, tpw):
        @pl.loop(0, tpw // IDX_CH)
        def _(b):
            pltpu.sync_copy(
                idx_hbm.at[pl.ds(sid * tpw + b * IDX_CH, IDX_CH)], idx_v)
            pltpu.sync_copy(
                seg_hbm.at[pl.ds(sid * tpw + b * IDX_CH, IDX_CH)], seg_v)

            sh_buf = [[], []]
            for g in range(IDX_CH // GRP):
                buf = g % 2
                for h in sh_buf[buf]:
                    h.wait()
                gh = [pltpu.async_copy(src.at[idx_v.at[g * GRP + k]],
                                       rows_v.at[buf, k], gsem)
                      for k in range(GRP)]
                sh = []
                for k in range(GRP):
                    gh[k].wait()
                    sh.append(pltpu.async_copy(
                        rows_v.at[buf, k], dst.at[seg_v.at[g * GRP + k]],
                        ssem, add=True))
                sh_buf[buf] = sh
            for sh in sh_buf:
                for h in sh:
                    h.wait()

    @pl.loop(0, ZB_ROWS)
    def _(i):
        zbuf[i, :] = jnp.zeros((LANE,), jnp.float32)

    @pl.loop(0, NCH // NC)
    def _(k):
        c = cid * (NCH // NC) + k

        zh = [pltpu.async_copy(
                  zbuf, xe_s.at[pl.ds(sid * 3760 + z * ZB_ROWS, ZB_ROWS)], gsem)
              for z in range(10)]
        zh.append(pltpu.async_copy(zbuf, xv_s.at[pl.ds(sid * XV_TPW, ZB_ROWS)],
                                   gsem))
        zh.append(pltpu.async_copy(
            zbuf.at[pl.ds(0, XV_TPW - ZB_ROWS)],
            xv_s.at[pl.ds(sid * XV_TPW + ZB_ROWS, XV_TPW - ZB_ROWS)], gsem))
        zh.append(pltpu.async_copy(t_hbm.at[pl.ds(sid * NT_TPW, NT_TPW), c],
                                   tbl_s.at[pl.ds(sid * NT_TPW, NT_TPW)], gsem))
        for h in zh:
            h.wait()
        plsc.subcore_barrier()

        _seg_pass(tbl_s, i1_hbm, s1_hbm, xe_s, S1_TPW)
        plsc.subcore_barrier()

        _seg_pass(xe_s, e2_hbm, v2_hbm, xv_s, S2_TPW)
        plsc.subcore_barrier()

        pltpu.sync_copy(xv_s.at[pl.ds(sid * XV_TPW, XV_TPW)],
                        out_hbm.at[c].at[pl.ds(sid * XV_TPW, XV_TPW)])
        plsc.subcore_barrier()


def _sc_final_gather(ee_hbm, i6_hbm, rp_hbm, o6_hbm, or_hbm,
                     i6_v, rows_v):
    cid = lax.axis_index("c")
    sid = lax.axis_index("s")
    wid = sid * NC + cid

    pltpu.sync_copy(i6_hbm.at[wid], i6_v)

    @pl.loop(0, 6)
    def _(j):
        pltpu.sync_copy(ee_hbm.at[i6_v.at[j]], rows_v)
        pltpu.sync_copy(rows_v, o6_hbm.at[pl.ds((j * 32 + wid) * 128, 128)])

    pltpu.sync_copy(rp_hbm.at[i6_v.at[6]], rows_v)
    pltpu.sync_copy(rows_v, or_hbm.at[pl.ds(wid * 128, 128)])


@jax.jit
def _run(r_idx, e1_idx, e2_idx, e3_idx, e4_idx, e5_idx, e6_idx,
         V, E, ty, emb_E, emb_R, emb_ty, W_lin, b_lin, scale_lin, eps):
    f32 = jnp.float32

    X = pl.pallas_call(
        _lorentz_tc,
        out_shape=jax.ShapeDtypeStruct((N_ENT, D), f32),
    )(emb_E, W_lin, b_lin, jnp.reshape(scale_lin, (1, 1)).astype(f32))

    T = jnp.concatenate(
        [X, -emb_ty, jnp.zeros((NT_PAD - N_ENT - NTY, D), f32)], axis=0)
    Tc = T.reshape(NT_PAD, NCH, LANE)

    n1pad = S1_PAD - 2 * N_INC
    idx1 = jnp.concatenate(
        [V, N_ENT + ty, jnp.full((n1pad,), ZROW, jnp.int32)]).reshape(-1, 128)
    seg1 = jnp.concatenate(
        [E, E, jnp.zeros((n1pad,), jnp.int32)]).reshape(-1, 128)
    npad = S2_PAD - N_INC
    e2 = jnp.concatenate(
        [E, jnp.full((npad,), XE_PAD_ROW, jnp.int32)]).reshape(-1, 128)
    v2 = jnp.concatenate(
        [V, jnp.full((npad,), XV_PAD_ROW, jnp.int32)]).reshape(-1, 128)

    mesh = plsc.VectorSubcoreMesh(core_axis_name="c", subcore_axis_name="s")
    sc_params = pltpu.CompilerParams(use_tc_tiling_on_sc=False)
    xv_ch = pl.kernel(
        _sc_two_stage,
        out_type=jax.ShapeDtypeStruct((NCH, XV_ROWS, LANE), f32),
        mesh=mesh,
        compiler_params=sc_params,
        scratch_types=[
            pltpu.VMEM((IDX_CH, 128), jnp.int32),
            pltpu.VMEM((IDX_CH, 128), jnp.int32),
            pltpu.VMEM((2, GRP, 128, LANE), f32),
            pltpu.VMEM((ZB_ROWS, LANE), f32),
            pltpu.SemaphoreType.DMA,
            pltpu.SemaphoreType.DMA,
            pltpu.VMEM_SHARED((NT_PAD, LANE), f32),
            pltpu.VMEM_SHARED((XE_ROWS, LANE), f32),
            pltpu.VMEM_SHARED((XV_ROWS, LANE), f32),
        ],
    )(Tc, idx1, seg1, e2, v2)

    Xv = jnp.transpose(xv_ch, (1, 0, 2)).reshape(XV_ROWS, D)[:N_ENT]

    E_e = pl.pallas_call(
        _logmap_tc,
        out_shape=jax.ShapeDtypeStruct((N_ENT, D), f32),
    )(Xv, X, jnp.reshape(eps, (1, 1)).astype(f32))

    R_p = emb_R.at[0].set(jnp.ones((D,), f32))

    idx6 = jnp.transpose(
        jnp.stack([e1_idx, e2_idx, e3_idx, e4_idx, e5_idx, e6_idx]
                  ).reshape(6, 32, 128), (1, 0, 2))
    iall = jnp.concatenate(
        [idx6, r_idx.reshape(32, 1, 128),
         jnp.zeros((32, 1, 128), jnp.int32)], axis=1)
    g6, gr = pl.kernel(
        _sc_final_gather,
        out_type=(jax.ShapeDtypeStruct((6 * BATCH, D), f32),
                  jax.ShapeDtypeStruct((BATCH, D), f32)),
        mesh=mesh,
        scratch_types=[
            pltpu.VMEM((8, 128), jnp.int32),
            pltpu.VMEM((128, D), f32),
        ],
    )(E_e, iall, R_p)

    out = pl.pallas_call(
        _final_tc,
        out_shape=jax.ShapeDtypeStruct((BATCH // 128, 128), f32),
    )(g6.reshape(6, BATCH, D), gr)
    return out.reshape(BATCH)


def kernel(r_idx, e1_idx, e2_idx, e3_idx, e4_idx, e5_idx, e6_idx, ms, bs,
           V, E, ty, emb_E, emb_R, emb_ty, W_lin, b_lin, scale_lin, eps):
    del ms, bs
    return _run(r_idx, e1_idx, e2_idx, e3_idx, e4_idx, e5_idx, e6_idx,
                V, E, ty, emb_E, emb_R, emb_ty, W_lin, b_lin, scale_lin, eps)


# speedup vs baseline: 1.1581x; 1.0177x over previous
import functools
import jax
import jax.numpy as jnp
from jax import lax
from jax.experimental import pallas as pl
from jax.experimental.pallas import tpu as pltpu
from jax.experimental.pallas import tpu_sc as plsc

N_ENT = 10000
N_HE = 60000
N_INC = 320000
D = 128
BATCH = 4096
NTY = 294

NC = 2
NS = 16
LANE = 16

NCH = D // LANE
ZROW = N_ENT + NTY
NT_PAD = 10368
NT_TPW = NT_PAD // NS

S1_PAD = 16 * 40960
S1_TPW = S1_PAD // NS // 128
S2_PAD = 16 * 20480
S2_TPW = S2_PAD // NS // 128
IDX_CH = 40

XE_ROWS = 60160
XV_ROWS = 10112
XV_TPW = XV_ROWS // NS
XE_PAD_ROW = 60100
XV_PAD_ROW = 10050
ZB_ROWS = 376


def _lorentz_tc(emb_ref, w_ref, b_ref, s_ref, x_ref):
    x = jax.lax.dot_general(emb_ref[...], w_ref[...],
                            dimension_numbers=(((1,), (1,)), ((), ())),
                            preferred_element_type=jnp.float32)
    x = x + b_ref[...]
    col = lax.broadcasted_iota(jnp.int32, x.shape, 1)
    is0 = col == 0
    time = jax.nn.sigmoid(x[:, :1]) * jnp.exp(s_ref[0, 0]) + 1.1
    xs = jnp.where(is0, 0.0, x)
    denom = jnp.maximum(jnp.sum(xs * xs, axis=-1, keepdims=True), 1e-8)
    scale = (time * time - 1.0) / denom
    x_ref[...] = jnp.where(is0, time, x * jnp.sqrt(scale))


def _logmap_tc(xv_ref, x_ref, eps_ref, out_ref):
    xc = eps_ref[0, 0] * xv_ref[...] + x_ref[...]
    col = lax.broadcasted_iota(jnp.int32, xc.shape, 1)
    row = lax.broadcasted_iota(jnp.int32, xc.shape, 0)
    is0 = col == 0
    y = jnp.where(is0, 0.0, xc)
    y_norm = jnp.maximum(jnp.sqrt(jnp.sum(y * y, axis=-1, keepdims=True)), 1e-8)
    theta = jnp.maximum(xc[:, :1], 1.0 + 1e-7)
    acosh = jnp.log(theta + jnp.sqrt(theta * theta - 1.0))
    out = jnp.where(is0, 0.0, xc * (acosh / y_norm))
    out_ref[...] = jnp.where(row == 0, 1.0, out)


def _final_tc(g6_ref, gr_ref, out_ref):
    p = gr_ref[...]
    for j in range(6):
        p = p * g6_ref[j]
    out_ref[...] = jnp.sum(p, axis=-1).reshape(out_ref.shape)


GRP = 5


def _sc_two_stage(t_hbm, i1_hbm, s1_hbm, e2_hbm, v2_hbm, out_hbm,
                  idx_v, seg_v, rows_v, zbuf, gsem, ssem, psem,
                  tbl_s, xe_s, xv_s):
    cid = lax.axis_index("c")
    sid = lax.axis_index("s")

    def _seg_pass(src, idx_hbm, seg_hbm, dst, tpw):
        nblk = tpw // IDX_CH

        def load(b, buf):
            base = sid * tpw + b * IDX_CH
            return [pltpu.async_copy(idx_hbm.at[pl.ds(base, IDX_CH)],
                                     idx_v.at[buf], psem),
                    pltpu.async_copy(seg_hbm.at[pl.ds(base, IDX_CH)],
                                     seg_v.at[buf], psem)]

        ph = {0: load(0, 0)}
        for b in range(nblk):
            buf = b % 2
            if b + 1 < nblk:
                ph[1 - buf] = load(b + 1, 1 - buf)
            for h in ph[buf]:
                h.wait()
            sh_buf = [[], []]
            for g in range(IDX_CH // GRP):
                rbuf = g % 2
                for h in sh_buf[rbuf]:
                    h.wait()
                gh = [pltpu.async_copy(src.at[idx_v.at[buf, g * GRP + k]],
                                       rows_v.at[rbuf, k], gsem)
                      for k in range(GRP)]
                sh = []
                for k in range(GRP):
                    gh[k].wait()
                    sh.append(pltpu.async_copy(
                        rows_v.at[rbuf, k], dst.at[seg_v.at[buf, g * GRP + k]],
                        ssem, add=True))
                sh_buf[rbuf] = sh
            for sh in sh_buf:
                for h in sh:
                    h.wait()

    @pl.loop(0, ZB_ROWS)
    def _(i):
        zbuf[i, :] = jnp.zeros((LANE,), jnp.float32)

    @pl.loop(0, NCH // NC)
    def _(k):
        c = cid * (NCH // NC) + k

        zh = [pltpu.async_copy(
                  zbuf, xe_s.at[pl.ds(sid * 3760 + z * ZB_ROWS, ZB_ROWS)], gsem)
              for z in range(10)]
        zh.append(pltpu.async_copy(zbuf, xv_s.at[pl.ds(sid * XV_TPW, ZB_ROWS)],
                                   gsem))
        zh.append(pltpu.async_copy(
            zbuf.at[pl.ds(0, XV_TPW - ZB_ROWS)],
            xv_s.at[pl.ds(sid * XV_TPW + ZB_ROWS, XV_TPW - ZB_ROWS)], gsem))
        zh.append(pltpu.async_copy(t_hbm.at[pl.ds(sid * NT_TPW, NT_TPW), c],
                                   tbl_s.at[pl.ds(sid * NT_TPW, NT_TPW)], gsem))
        for h in zh:
            h.wait()
        plsc.subcore_barrier()

        _seg_pass(tbl_s, i1_hbm, s1_hbm, xe_s, S1_TPW)
        plsc.subcore_barrier()

        _seg_pass(xe_s, e2_hbm, v2_hbm, xv_s, S2_TPW)
        plsc.subcore_barrier()

        pltpu.sync_copy(xv_s.at[pl.ds(sid * XV_TPW, XV_TPW)],
                        out_hbm.at[c].at[pl.ds(sid * XV_TPW, XV_TPW)])
        plsc.subcore_barrier()


def _sc_final_gather(ee_hbm, i6_hbm, rp_hbm, o6_hbm, or_hbm,
                     i6_v, rows_v):
    cid = lax.axis_index("c")
    sid = lax.axis_index("s")
    wid = sid * NC + cid

    pltpu.sync_copy(i6_hbm.at[wid], i6_v)

    @pl.loop(0, 6)
    def _(j):
        pltpu.sync_copy(ee_hbm.at[i6_v.at[j]], rows_v)
        pltpu.sync_copy(rows_v, o6_hbm.at[pl.ds((j * 32 + wid) * 128, 128)])

    pltpu.sync_copy(rp_hbm.at[i6_v.at[6]], rows_v)
    pltpu.sync_copy(rows_v, or_hbm.at[pl.ds(wid * 128, 128)])


@jax.jit
def _run(r_idx, e1_idx, e2_idx, e3_idx, e4_idx, e5_idx, e6_idx,
         V, E, ty, emb_E, emb_R, emb_ty, W_lin, b_lin, scale_lin, eps):
    f32 = jnp.float32

    X = pl.pallas_call(
        _lorentz_tc,
        out_shape=jax.ShapeDtypeStruct((N_ENT, D), f32),
    )(emb_E, W_lin, b_lin, jnp.reshape(scale_lin, (1, 1)).astype(f32))

    T = jnp.concatenate(
        [X, -emb_ty, jnp.zeros((NT_PAD - N_ENT - NTY, D), f32)], axis=0)
    Tc = T.reshape(NT_PAD, NCH, LANE)

    n1pad = S1_PAD - 2 * N_INC
    idx1 = jnp.concatenate(
        [V, N_ENT + ty, jnp.full((n1pad,), ZROW, jnp.int32)]).reshape(-1, 128)
    seg1 = jnp.concatenate(
        [E, E, jnp.zeros((n1pad,), jnp.int32)]).reshape(-1, 128)
    npad = S2_PAD - N_INC
    e2 = jnp.concatenate(
        [E, jnp.full((npad,), XE_PAD_ROW, jnp.int32)]).reshape(-1, 128)
    v2 = jnp.concatenate(
        [V, jnp.full((npad,), XV_PAD_ROW, jnp.int32)]).reshape(-1, 128)

    mesh = plsc.VectorSubcoreMesh(core_axis_name="c", subcore_axis_name="s")
    sc_params = pltpu.CompilerParams(use_tc_tiling_on_sc=False)
    xv_ch = pl.kernel(
        _sc_two_stage,
        out_type=jax.ShapeDtypeStruct((NCH, XV_ROWS, LANE), f32),
        mesh=mesh,
        compiler_params=sc_params,
        scratch_types=[
            pltpu.VMEM((2, IDX_CH, 128), jnp.int32),
            pltpu.VMEM((2, IDX_CH, 128), jnp.int32),
            pltpu.VMEM((2, GRP, 128, LANE), f32),
            pltpu.VMEM((ZB_ROWS, LANE), f32),
            pltpu.SemaphoreType.DMA,
            pltpu.SemaphoreType.DMA,
            pltpu.SemaphoreType.DMA,
            pltpu.VMEM_SHARED((NT_PAD, LANE), f32),
            pltpu.VMEM_SHARED((XE_ROWS, LANE), f32),
            pltpu.VMEM_SHARED((XV_ROWS, LANE), f32),
        ],
    )(Tc, idx1, seg1, e2, v2)

    Xv = jnp.transpose(xv_ch, (1, 0, 2)).reshape(XV_ROWS, D)[:N_ENT]

    E_e = pl.pallas_call(
        _logmap_tc,
        out_shape=jax.ShapeDtypeStruct((N_ENT, D), f32),
    )(Xv, X, jnp.reshape(eps, (1, 1)).astype(f32))

    R_p = emb_R.at[0].set(jnp.ones((D,), f32))

    idx6 = jnp.transpose(
        jnp.stack([e1_idx, e2_idx, e3_idx, e4_idx, e5_idx, e6_idx]
                  ).reshape(6, 32, 128), (1, 0, 2))
    iall = jnp.concatenate(
        [idx6, r_idx.reshape(32, 1, 128),
         jnp.zeros((32, 1, 128), jnp.int32)], axis=1)
    g6, gr = pl.kernel(
        _sc_final_gather,
        out_type=(jax.ShapeDtypeStruct((6 * BATCH, D), f32),
                  jax.ShapeDtypeStruct((BATCH, D), f32)),
        mesh=mesh,
        scratch_types=[
            pltpu.VMEM((8, 128), jnp.int32),
            pltpu.VMEM((128, D), f32),
        ],
    )(E_e, iall, R_p)

    out = pl.pallas_call(
        _final_tc,
        out_shape=jax.ShapeDtypeStruct((BATCH // 128, 128), f32),
    )(g6.reshape(6, BATCH, D), gr)
    return out.reshape(BATCH)


def kernel(r_idx, e1_idx, e2_idx, e3_idx, e4_idx, e5_idx, e6_idx, ms, bs,
           V, E, ty, emb_E, emb_R, emb_ty, W_lin, b_lin, scale_lin, eps):
    del ms, bs
    return _run(r_idx, e1_idx, e2_idx, e3_idx, e4_idx, e5_idx, e6_idx,
                V, E, ty, emb_E, emb_R, emb_ty, W_lin, b_lin, scale_lin, eps)
